# baseline (speedup 1.0000x reference)
import functools

import jax
import jax.numpy as jnp
from jax import lax
from jax.experimental import pallas as pl
from jax.experimental.pallas import tpu as pltpu
from jax.experimental.pallas import tpu_sc as plsc

B = 16384
D = 32
SUP = 4
L = 16

_info = plsc.get_sparse_core_info()
_NC, _NS = _info.num_cores, _info.num_subcores
NW = _NC * _NS
BPW = B // NW
CH = 128
NCH = BPW // CH


def _sc_gather(users, items, mf_u, mf_i, mlp_u, mlp_i):
    mesh = plsc.VectorSubcoreMesh(core_axis_name="c", subcore_axis_name="s")

    @functools.partial(
        pl.kernel, mesh=mesh,
        out_type=jax.ShapeDtypeStruct((B, 4 * D), jnp.float32),
        scratch_types=[
            pltpu.VMEM((BPW,), jnp.int32),
            pltpu.VMEM((BPW,), jnp.int32),
            pltpu.VMEM((BPW,), jnp.int32),
            pltpu.VMEM((BPW,), jnp.int32),
            pltpu.VMEM((CH, 4 * D), jnp.float32),
            pltpu.VMEM((CH, 4 * D), jnp.float32),
            pltpu.VMEM((CH, 4 * D), jnp.float32),
            pltpu.VMEM((CH, 4 * D), jnp.float32),
            pltpu.VMEM((CH, 4 * D), jnp.float32),
            pltpu.SemaphoreType.DMA,
        ],
        compiler_params=pltpu.CompilerParams(use_tc_tiling_on_sc=True),
    )
    def k(users_h, items_h, mfu_h, mfi_h, mlpu_h, mlpi_h, out_h,
          u_v, i_v, su_v, si_v, g_mfu, g_mfi, g_mlpu, g_mlpi,
          stage, sem):
        wid = lax.axis_index("s") * _NC + lax.axis_index("c")
        base = wid * BPW
        pltpu.sync_copy(users_h.at[pl.ds(base, BPW)], u_v)
        pltpu.sync_copy(items_h.at[pl.ds(base, BPW)], i_v)

        def idx_body(g, _):
            sl = pl.ds(g * L, L)
            su_v[sl] = jnp.right_shift(u_v[sl], 2)
            si_v[sl] = jnp.right_shift(i_v[sl], 2)
            return 0

        lax.fori_loop(0, BPW // L, idx_body, 0, unroll=4)

        for c in range(NCH):
            isl = pl.ds(c * CH, CH)
            cps = [
                pltpu.async_copy(mfu_h.at[su_v.at[isl]], g_mfu, sem),
                pltpu.async_copy(mfi_h.at[si_v.at[isl]], g_mfi, sem),
                pltpu.async_copy(mlpu_h.at[su_v.at[isl]], g_mlpu, sem),
                pltpu.async_copy(mlpi_h.at[si_v.at[isl]], g_mlpi, sem),
            ]
            for cp in cps:
                cp.wait()

            def grp_body(g, _):
                uoffv = (u_v[pl.ds(c * CH + g * L, L)] & 3) * D
                ioffv = (i_v[pl.ds(c * CH + g * L, L)] & 3) * D
                for l in range(L):
                    r = g * L + l
                    off_u = uoffv[l]
                    off_i = ioffv[l]
                    for j in range(D // L):
                        stage[r, pl.ds(0 * D + j * L, L)] = g_mfu[r, pl.ds(off_u + j * L, L)]
                        stage[r, pl.ds(1 * D + j * L, L)] = g_mfi[r, pl.ds(off_i + j * L, L)]
                        stage[r, pl.ds(2 * D + j * L, L)] = g_mlpu[r, pl.ds(off_u + j * L, L)]
                        stage[r, pl.ds(3 * D + j * L, L)] = g_mlpi[r, pl.ds(off_i + j * L, L)]
                return 0

            lax.fori_loop(0, CH // L, grp_body, 0)
            pltpu.sync_copy(stage, out_h.at[pl.ds(base + c * CH, CH)])

    return k(users, items,
             mf_u.reshape(-1, 4 * D), mf_i.reshape(-1, 4 * D),
             mlp_u.reshape(-1, 4 * D), mlp_i.reshape(-1, 4 * D))


def _tc_body(emb, w1a, w1b, b1r, w2, b2r, w3, b3r, woa, wob, bor, out):
    f32 = jnp.float32
    e = emb[:]
    gmf = e[:, :D] * e[:, D:2 * D]
    h = jnp.dot(e[:, 2 * D:3 * D], w1a[:], preferred_element_type=f32)
    h = h + jnp.dot(e[:, 3 * D:], w1b[:], preferred_element_type=f32)
    h = jnp.maximum(h + b1r[:], 0.0)
    h = jnp.maximum(jnp.dot(h, w2[:], preferred_element_type=f32) + b2r[:], 0.0)
    h = jnp.maximum(jnp.dot(h, w3[:], preferred_element_type=f32) + b3r[:], 0.0)
    logit = (jnp.dot(gmf, woa[:], preferred_element_type=f32)
             + jnp.dot(h, wob[:], preferred_element_type=f32) + bor[:])
    out[:] = jax.nn.sigmoid(logit)


def _tc_mlp(emb, W1, b1, W2, b2, W3, b3, Wo, bo):
    bs = 2048
    grid = (B // bs,)
    w1a, w1b = W1[:D], W1[D:]
    woa, wob = Wo[:D], Wo[D:]
    b1r = b1.reshape(1, -1)
    b2r = b2.reshape(1, -1)
    b3r = b3.reshape(1, -1)
    bor = bo.reshape(1, 1)

    def full(a):
        return pl.BlockSpec(a.shape, lambda i: (0,) * a.ndim)

    return pl.pallas_call(
        _tc_body,
        grid=grid,
        in_specs=[
            pl.BlockSpec((bs, 4 * D), lambda i: (i, 0)),
            full(w1a), full(w1b), full(b1r),
            full(W2), full(b2r),
            full(W3), full(b3r),
            full(woa), full(wob), full(bor),
        ],
        out_specs=pl.BlockSpec((bs, 1), lambda i: (i, 0)),
        out_shape=jax.ShapeDtypeStruct((B, 1), jnp.float32),
    )(emb, w1a, w1b, b1r, W2, b2r, W3, b3r, woa, wob, bor)


def kernel(users, items, mf_u, mf_i, mlp_u, mlp_i, W1, b1, W2, b2, W3, b3,
           Wo, bo):
    emb = _sc_gather(users, items, mf_u, mf_i, mlp_u, mlp_i)
    return _tc_mlp(emb, W1, b1, W2, b2, W3, b3, Wo, bo)


# speedup vs baseline: 3.2796x; 3.2796x over previous
import functools

import jax
import jax.numpy as jnp
from jax import lax
from jax.experimental import pallas as pl
from jax.experimental.pallas import tpu as pltpu
from jax.experimental.pallas import tpu_sc as plsc

B = 16384
D = 32
F = 4 * D
L = 16

_info = plsc.get_sparse_core_info()
_NC, _NS = _info.num_cores, _info.num_subcores
NW = _NC * _NS
BPW = B // NW
NG = BPW // L


def _sc_gather(users, items, mf_uT, mf_iT, mlp_uT, mlp_iT):
    mesh = plsc.VectorSubcoreMesh(core_axis_name="c", subcore_axis_name="s")

    @functools.partial(
        pl.kernel, mesh=mesh,
        out_type=jax.ShapeDtypeStruct((B, F), jnp.float32),
        scratch_types=[
            pltpu.VMEM((BPW,), jnp.int32),
            pltpu.VMEM((BPW,), jnp.int32),
            pltpu.VMEM((4, D, 128), jnp.float32),
            pltpu.VMEM((4, D, 128), jnp.float32),
            pltpu.VMEM((4, D, 128), jnp.float32),
            pltpu.VMEM((4, D, 128), jnp.float32),
            pltpu.VMEM((L, F), jnp.float32),
            pltpu.SemaphoreType.DMA,
        ],
        compiler_params=pltpu.CompilerParams(
            use_tc_tiling_on_sc=True, needs_layout_passes=False),
    )
    def k(users_h, items_h, mfu_h, mfi_h, mlpu_h, mlpi_h, out_h,
          u_v, i_v, b_mfu, b_mfi, b_mlpu, b_mlpi, stage, sem):
        wid = lax.axis_index("s") * _NC + lax.axis_index("c")
        base = wid * BPW
        pltpu.sync_copy(users_h.at[pl.ds(base, BPW)], u_v)
        pltpu.sync_copy(items_h.at[pl.ds(base, BPW)], i_v)

        tabs = [(mfu_h, b_mfu, 0), (mfi_h, b_mfi, 1),
                (mlpu_h, b_mlpu, 2), (mlpi_h, b_mlpi, 3)]
        iot = lax.iota(jnp.int32, L)

        def grp_body(g, _):
            ug = u_v[pl.ds(g * L, L)]
            ig = i_v[pl.ds(g * L, L)]
            ucol = (ug >> 7) << 7
            icol = (ig >> 7) << 7
            ulane = ug & 127
            ilane = ig & 127
            for sub in range(4):
                cps = []
                for j in range(4):
                    l = sub * 4 + j
                    for tab, blk, t in tabs:
                        col = ucol[l] if t in (0, 2) else icol[l]
                        col = pl.multiple_of(col, 128)
                        cps.append(pltpu.async_copy(
                            tab.at[:, pl.ds(col, 128)], blk.at[j], sem))
                for cp in cps:
                    cp.wait()
                for j in range(4):
                    l = sub * 4 + j
                    for tab, blk, t in tabs:
                        lane = ulane[l] if t in (0, 2) else ilane[l]
                        lvec = jnp.full((L,), lane, dtype=jnp.int32)
                        nvec = jnp.full((L,), j, dtype=jnp.int32)
                        for h in range(D // L):
                            vals = plsc.load_gather(
                                blk, [nvec, h * L + iot, lvec])
                            stage[l, pl.ds(t * D + h * L, L)] = vals
            pltpu.sync_copy(stage, out_h.at[pl.ds(base + g * L, L)])
            return 0

        lax.fori_loop(0, NG, grp_body, 0)

    return k(users, items, mf_uT, mf_iT, mlp_uT, mlp_iT)


def _tc_body(emb, w1a, w1b, b1r, w2, b2r, w3, b3r, woa, wob, bor, out):
    f32 = jnp.float32
    e = emb[:]
    gmf = e[:, :D] * e[:, D:2 * D]
    h = jnp.dot(e[:, 2 * D:3 * D], w1a[:], preferred_element_type=f32)
    h = h + jnp.dot(e[:, 3 * D:], w1b[:], preferred_element_type=f32)
    h = jnp.maximum(h + b1r[:], 0.0)
    h = jnp.maximum(jnp.dot(h, w2[:], preferred_element_type=f32) + b2r[:], 0.0)
    h = jnp.maximum(jnp.dot(h, w3[:], preferred_element_type=f32) + b3r[:], 0.0)
    logit = (jnp.dot(gmf, woa[:], preferred_element_type=f32)
             + jnp.dot(h, wob[:], preferred_element_type=f32) + bor[:])
    out[:] = jax.nn.sigmoid(logit)


def _tc_mlp(emb, W1, b1, W2, b2, W3, b3, Wo, bo):
    bs = 2048
    grid = (B // bs,)
    w1a, w1b = W1[:D], W1[D:]
    woa, wob = Wo[:D], Wo[D:]
    b1r = b1.reshape(1, -1)
    b2r = b2.reshape(1, -1)
    b3r = b3.reshape(1, -1)
    bor = bo.reshape(1, 1)

    def full(a):
        return pl.BlockSpec(a.shape, lambda i: (0,) * a.ndim)

    return pl.pallas_call(
        _tc_body,
        grid=grid,
        in_specs=[
            pl.BlockSpec((bs, F), lambda i: (i, 0)),
            full(w1a), full(w1b), full(b1r),
            full(W2), full(b2r),
            full(W3), full(b3r),
            full(woa), full(wob), full(bor),
        ],
        out_specs=pl.BlockSpec((bs, 1), lambda i: (i, 0)),
        out_shape=jax.ShapeDtypeStruct((B, 1), jnp.float32),
    )(emb, w1a, w1b, b1r, W2, b2r, W3, b3r, woa, wob, bor)


def kernel(users, items, mf_u, mf_i, mlp_u, mlp_i, W1, b1, W2, b2, W3, b3,
           Wo, bo):
    emb = _sc_gather(users, items, mf_u.T, mf_i.T, mlp_u.T, mlp_i.T)
    return _tc_mlp(emb, W1, b1, W2, b2, W3, b3, Wo, bo)
